# baseline (speedup 1.0000x reference)
import functools

import jax
import jax.numpy as jnp
from jax import lax
from jax.experimental import pallas as pl
from jax.experimental.pallas import tpu as pltpu
from jax.experimental.pallas import tpu_sc as plsc

_LANES = 16


def _sc_gather_pairs(pairs, idx):
    B = idx.shape[0]
    D2 = pairs.shape[1]
    info = plsc.get_sparse_core_info()
    nc, ns = info.num_cores, info.num_subcores
    nw = nc * ns
    b_per_w = B // nw
    mesh = plsc.VectorSubcoreMesh(core_axis_name="c", subcore_axis_name="s")

    @functools.partial(
        pl.kernel,
        mesh=mesh,
        out_type=jax.ShapeDtypeStruct((B, D2), jnp.float32),
        scratch_types=[
            pltpu.VMEM((b_per_w,), jnp.int32),
            pltpu.VMEM((b_per_w,), jnp.int32),
            pltpu.VMEM((b_per_w, D2), jnp.float32),
            pltpu.SemaphoreType.DMA,
        ],
    )
    def k(pairs_hbm, idx_hbm, out_hbm, idx_v, t_v, rows_v, sem):
        wid = lax.axis_index("s") * nc + lax.axis_index("c")
        base = wid * b_per_w
        pltpu.sync_copy(idx_hbm.at[pl.ds(base, b_per_w)], idx_v)

        def halve(g, _):
            v = idx_v[pl.ds(g * _LANES, _LANES)]
            t_v[pl.ds(g * _LANES, _LANES)] = lax.shift_right_logical(v, 1)
            return _

        lax.fori_loop(0, b_per_w // _LANES, halve, 0)
        pltpu.async_copy(pairs_hbm.at[t_v], rows_v, sem).wait()
        pltpu.sync_copy(rows_v, out_hbm.at[pl.ds(base, b_per_w)])

    return k(pairs, idx)


def _tc_dense(pair_rows, idx_col, mean, W1, b1):
    B, D = mean.shape
    blk = 2048

    def body(e_ref, i_ref, m_ref, w_ref, b_ref, o_ref):
        par = lax.rem(i_ref[...], 2) == 1
        x = jnp.where(par, e_ref[:, D:], e_ref[:, :D]) + m_ref[...]
        y = lax.dot_general(
            x, w_ref[...], (((1,), (1,)), ((), ())),
            preferred_element_type=jnp.float32,
        )
        y = y + b_ref[...]
        o_ref[...] = jnp.where(y >= 0, y, 0.2 * y)

    return pl.pallas_call(
        body,
        grid=(B // blk,),
        in_specs=[
            pl.BlockSpec((blk, 2 * D), lambda i: (i, 0)),
            pl.BlockSpec((blk, 1), lambda i: (i, 0)),
            pl.BlockSpec((blk, D), lambda i: (i, 0)),
            pl.BlockSpec((D, D), lambda i: (0, 0)),
            pl.BlockSpec((1, D), lambda i: (0, 0)),
        ],
        out_specs=pl.BlockSpec((blk, D), lambda i: (i, 0)),
        out_shape=jax.ShapeDtypeStruct((B, D), jnp.float32),
    )(pair_rows, idx_col, mean, W1, b1.reshape(1, D))


def kernel(user_idx, mean_poi_embeddings, user_embedding, W1, b1):
    idx = user_idx.astype(jnp.int32)
    V, D = user_embedding.shape
    pairs = user_embedding.reshape(V // 2, 2 * D)
    pair_rows = _sc_gather_pairs(pairs, idx)
    return _tc_dense(pair_rows, idx.reshape(-1, 1), mean_poi_embeddings, W1, b1)


# speedup vs baseline: 1.7037x; 1.7037x over previous
import functools

import jax
import jax.numpy as jnp
from jax import lax
from jax.experimental import pallas as pl
from jax.experimental.pallas import tpu as pltpu
from jax.experimental.pallas import tpu_sc as plsc

_LANES = 16


def _sc_gather(table, idx):
    B = idx.shape[0]
    D = table.shape[1]
    info = plsc.get_sparse_core_info()
    nc, ns = info.num_cores, info.num_subcores
    nw = nc * ns
    b_per_w = B // nw
    mesh = plsc.VectorSubcoreMesh(core_axis_name="c", subcore_axis_name="s")

    @functools.partial(
        pl.kernel,
        mesh=mesh,
        out_type=jax.ShapeDtypeStruct((B, D), jnp.float32),
        scratch_types=[
            pltpu.VMEM((b_per_w,), jnp.int32),
            pltpu.VMEM((b_per_w, D), jnp.float32),
            pltpu.SemaphoreType.DMA,
        ],
        compiler_params=pltpu.CompilerParams(use_tc_tiling_on_sc=True),
    )
    def k(table_hbm, idx_hbm, out_hbm, idx_v, rows_v, sem):
        wid = lax.axis_index("s") * nc + lax.axis_index("c")
        base = wid * b_per_w
        pltpu.sync_copy(idx_hbm.at[pl.ds(base, b_per_w)], idx_v)

        def issue(g, _):
            v = idx_v[pl.ds(g * _LANES, _LANES)]
            for l in range(_LANES):
                r = v[l]
                pltpu.make_async_copy(
                    table_hbm.at[pl.ds(r, 1)],
                    rows_v.at[pl.ds(g * _LANES + l, 1)],
                    sem,
                ).start()
            return _

        lax.fori_loop(0, b_per_w // _LANES, issue, 0)
        pltpu.make_async_copy(
            table_hbm.at[pl.ds(0, b_per_w)], rows_v, sem
        ).wait()
        pltpu.sync_copy(rows_v, out_hbm.at[pl.ds(base, b_per_w)])

    return k(table, idx)


def _tc_dense(embed, mean, W1, b1):
    B, D = embed.shape
    blk = 2048

    def body(e_ref, m_ref, w_ref, b_ref, o_ref):
        x = e_ref[...] + m_ref[...]
        y = lax.dot_general(
            x, w_ref[...], (((1,), (1,)), ((), ())),
            preferred_element_type=jnp.float32,
        )
        y = y + b_ref[...]
        o_ref[...] = jnp.where(y >= 0, y, 0.2 * y)

    return pl.pallas_call(
        body,
        grid=(B // blk,),
        in_specs=[
            pl.BlockSpec((blk, D), lambda i: (i, 0)),
            pl.BlockSpec((blk, D), lambda i: (i, 0)),
            pl.BlockSpec((D, D), lambda i: (0, 0)),
            pl.BlockSpec((1, D), lambda i: (0, 0)),
        ],
        out_specs=pl.BlockSpec((blk, D), lambda i: (i, 0)),
        out_shape=jax.ShapeDtypeStruct((B, D), jnp.float32),
    )(embed, mean, W1, b1.reshape(1, D))


def kernel(user_idx, mean_poi_embeddings, user_embedding, W1, b1):
    idx = user_idx.astype(jnp.int32)
    embed = _sc_gather(user_embedding, idx)
    return _tc_dense(embed, mean_poi_embeddings, W1, b1)


# speedup vs baseline: 1.7514x; 1.0280x over previous
import functools

import jax
import jax.numpy as jnp
from jax import lax
from jax.experimental import pallas as pl
from jax.experimental.pallas import tpu as pltpu
from jax.experimental.pallas import tpu_sc as plsc

_LANES = 16


def _sc_gather(table, idx):
    B = idx.shape[0]
    D = table.shape[1]
    info = plsc.get_sparse_core_info()
    nc, ns = info.num_cores, info.num_subcores
    nw = nc * ns
    b_per_w = B // nw
    mesh = plsc.VectorSubcoreMesh(core_axis_name="c", subcore_axis_name="s")

    @functools.partial(
        pl.kernel,
        mesh=mesh,
        out_type=jax.ShapeDtypeStruct((B, D), jnp.float32),
        scratch_types=[
            pltpu.VMEM((b_per_w,), jnp.int32),
            pltpu.VMEM((b_per_w, D), jnp.float32),
            pltpu.SemaphoreType.DMA,
        ],
        compiler_params=pltpu.CompilerParams(use_tc_tiling_on_sc=True),
    )
    def k(table_hbm, idx_hbm, out_hbm, idx_v, rows_v, sem):
        wid = lax.axis_index("s") * nc + lax.axis_index("c")
        base = wid * b_per_w
        pltpu.sync_copy(idx_hbm.at[pl.ds(base, b_per_w)], idx_v)

        def issue(g, _):
            v = idx_v[pl.ds(g * _LANES, _LANES)]
            for l in range(_LANES):
                r = v[l]
                pltpu.make_async_copy(
                    table_hbm.at[pl.ds(r, 1)],
                    rows_v.at[pl.ds(g * _LANES + l, 1)],
                    sem,
                ).start()
            return _

        lax.fori_loop(0, b_per_w // _LANES, issue, 0)
        pltpu.make_async_copy(
            table_hbm.at[pl.ds(0, b_per_w)], rows_v, sem
        ).wait()
        pltpu.sync_copy(rows_v, out_hbm.at[pl.ds(base, b_per_w)])

    return k(table, idx)


def _tc_dense_t(embed, mean_t, W1, b1):
    D, B = mean_t.shape
    blk = 2048

    def body(e_ref, m_ref, w_ref, b_ref, o_ref):
        y = lax.dot_general(
            w_ref[...], e_ref[...], (((1,), (1,)), ((), ())),
            preferred_element_type=jnp.float32,
        )
        y = y + lax.dot_general(
            w_ref[...], m_ref[...], (((1,), (0,)), ((), ())),
            preferred_element_type=jnp.float32,
        )
        y = y + b_ref[...]
        o_ref[...] = jnp.where(y >= 0, y, 0.2 * y)

    return pl.pallas_call(
        body,
        grid=(B // blk,),
        in_specs=[
            pl.BlockSpec((blk, D), lambda i: (i, 0)),
            pl.BlockSpec((D, blk), lambda i: (0, i)),
            pl.BlockSpec((D, D), lambda i: (0, 0)),
            pl.BlockSpec((D, 1), lambda i: (0, 0)),
        ],
        out_specs=pl.BlockSpec((D, blk), lambda i: (0, i)),
        out_shape=jax.ShapeDtypeStruct((D, B), jnp.float32),
    )(embed, mean_t, W1, b1.reshape(D, 1))


def kernel(user_idx, mean_poi_embeddings, user_embedding, W1, b1):
    idx = user_idx.astype(jnp.int32)
    embed = _sc_gather(user_embedding, idx)
    out_t = _tc_dense_t(embed, mean_poi_embeddings.T, W1, b1)
    return out_t.T
